# baseline (speedup 1.0000x reference)
import jax
import jax.numpy as jnp
from jax.experimental import pallas as pl
from jax.experimental.pallas import tpu as pltpu

TOP_K = 8
NUM_EXPERTS = 64
D_MODEL = 2048

BLOCK_TOKENS = 1024


def _gating_kernel(x_ref, w_ref, bt_ref, topw_ref, topi_ref, weights_ref):
    lt = jax.lax.dot_general(
        w_ref[...], x_ref[...],
        dimension_numbers=(((1,), (1,)), ((), ())),
        preferred_element_type=jnp.float32,
    ) + bt_ref[...]
    et = jnp.exp(lt)
    st = jnp.sum(et, axis=0, keepdims=True)
    rst = 1.0 / st
    weights_ref[...] = (et * rst).T

    rows = jax.lax.broadcasted_iota(jnp.int32, et.shape, 0)
    bits = jax.lax.bitcast_convert_type(et, jnp.int32)
    keys = jax.lax.bitcast_convert_type(
        (bits & ~0x3F) | (NUM_EXPERTS - 1 - rows), jnp.float32)
    picked = []
    for k in range(TOP_K):
        kmax = jnp.max(keys, axis=0, keepdims=True)
        picked.append(kmax)
        if k + 1 < TOP_K:
            keys = jnp.where(keys == kmax, 0.0, keys)
    kcat = jax.lax.bitcast_convert_type(jnp.concatenate(picked, axis=0),
                                        jnp.int32)
    topi_ref[...] = ((NUM_EXPERTS - 1) - (kcat & 0x3F)).T
    e_sel = jax.lax.bitcast_convert_type((kcat & ~0x3F) | 0x20, jnp.float32)
    topw_ref[...] = (e_sel * rst).T


def kernel(x, W, b):
    n_tokens = x.shape[0]
    grid = (n_tokens // BLOCK_TOKENS,)
    bt = b.reshape(NUM_EXPERTS, 1)
    topw, topi, weights = pl.pallas_call(
        _gating_kernel,
        grid=grid,
        in_specs=[
            pl.BlockSpec((BLOCK_TOKENS, D_MODEL), lambda i: (i, 0)),
            pl.BlockSpec((NUM_EXPERTS, D_MODEL), lambda i: (0, 0)),
            pl.BlockSpec((NUM_EXPERTS, 1), lambda i: (0, 0)),
        ],
        out_specs=[
            pl.BlockSpec((BLOCK_TOKENS, TOP_K), lambda i: (i, 0)),
            pl.BlockSpec((BLOCK_TOKENS, TOP_K), lambda i: (i, 0)),
            pl.BlockSpec((BLOCK_TOKENS, NUM_EXPERTS), lambda i: (i, 0)),
        ],
        out_shape=[
            jax.ShapeDtypeStruct((n_tokens, TOP_K), jnp.float32),
            jax.ShapeDtypeStruct((n_tokens, TOP_K), jnp.int32),
            jax.ShapeDtypeStruct((n_tokens, NUM_EXPERTS), jnp.float32),
        ],
        compiler_params=pltpu.CompilerParams(
            dimension_semantics=(pltpu.PARALLEL,),
        ),
    )(x, W, bt)
    return topw, topi, weights


# speedup vs baseline: 1.0065x; 1.0065x over previous
import jax
import jax.numpy as jnp
from jax.experimental import pallas as pl
from jax.experimental.pallas import tpu as pltpu

TOP_K = 8
NUM_EXPERTS = 64
D_MODEL = 2048

BLOCK_TOKENS = 2048


def _gating_kernel(x_ref, w_ref, bt_ref, topw_ref, topi_ref, weights_ref):
    lt = jax.lax.dot_general(
        w_ref[...], x_ref[...],
        dimension_numbers=(((1,), (1,)), ((), ())),
        preferred_element_type=jnp.float32,
    ) + bt_ref[...]
    et = jnp.exp(lt)
    st = jnp.sum(et, axis=0, keepdims=True)
    rst = 1.0 / st
    weights_ref[...] = (et * rst).T

    rowsf = jax.lax.broadcasted_iota(jnp.int32, et.shape, 0).astype(
        jnp.float32)
    work = et
    picked_w, picked_i = [], []
    for k in range(TOP_K):
        kmax = jnp.max(work, axis=0, keepdims=True)
        idxf = jnp.min(jnp.where(work == kmax, rowsf, float(NUM_EXPERTS)),
                       axis=0, keepdims=True)
        picked_w.append(kmax)
        picked_i.append(idxf)
        if k + 1 < TOP_K:
            work = jnp.where(rowsf == idxf, -1.0, work)
    wcat = jnp.concatenate(picked_w, axis=0)
    icat = jnp.concatenate(picked_i, axis=0)
    topi_ref[...] = icat.astype(jnp.int32).T
    topw_ref[...] = (wcat * rst).T


def kernel(x, W, b):
    n_tokens = x.shape[0]
    grid = (n_tokens // BLOCK_TOKENS,)
    bt = b.reshape(NUM_EXPERTS, 1)
    topw, topi, weights = pl.pallas_call(
        _gating_kernel,
        grid=grid,
        in_specs=[
            pl.BlockSpec((BLOCK_TOKENS, D_MODEL), lambda i: (i, 0)),
            pl.BlockSpec((NUM_EXPERTS, D_MODEL), lambda i: (0, 0)),
            pl.BlockSpec((NUM_EXPERTS, 1), lambda i: (0, 0)),
        ],
        out_specs=[
            pl.BlockSpec((BLOCK_TOKENS, TOP_K), lambda i: (i, 0)),
            pl.BlockSpec((BLOCK_TOKENS, TOP_K), lambda i: (i, 0)),
            pl.BlockSpec((BLOCK_TOKENS, NUM_EXPERTS), lambda i: (i, 0)),
        ],
        out_shape=[
            jax.ShapeDtypeStruct((n_tokens, TOP_K), jnp.float32),
            jax.ShapeDtypeStruct((n_tokens, TOP_K), jnp.int32),
            jax.ShapeDtypeStruct((n_tokens, NUM_EXPERTS), jnp.float32),
        ],
        compiler_params=pltpu.CompilerParams(
            dimension_semantics=(pltpu.PARALLEL,),
        ),
    )(x, W, bt)
    return topw, topi, weights
